# baseline (speedup 1.0000x reference)
import numpy as np
import jax
import jax.numpy as jnp
from jax import lax
from jax.experimental import pallas as pl
from jax.experimental.pallas import tpu as pltpu
from jax.experimental.pallas import tpu_sc as plsc

_D = 64
_B = 16
_L = 2048
_NC = 2
_NS = 16
_NW = _NC * _NS
_PW = _L // _NW
_RW = _B * _PW
_GC = 128
_NG = _RW // _GC
_SCALE = 8.0


def _pos_encoding_np():
    pos = np.arange(_L)[:, None]
    i = np.arange(_D)[None, :]
    angle_rates = 1.0 / np.power(10000, 2 * (i // 2) / np.float32(_D))
    angle_rads = pos * angle_rates
    angle_rads[:, 0::2] = np.sin(angle_rads[:, 0::2])
    angle_rads[:, 1::2] = np.cos(angle_rads[:, 1::2])
    return angle_rads.astype(np.float32)


_POS = _pos_encoding_np()


def _sc_body(x_hbm, table_hbm, pos_hbm, out_hbm,
             idx_v, rows_v, pos_v, sem_in, sem_g, sem_out):
    wid = lax.axis_index("s") * _NC + lax.axis_index("c")
    p0 = pl.multiple_of(wid * _PW, _PW)

    copies = []
    for b in range(_B):
        copies.append(pltpu.async_copy(
            x_hbm.at[b, pl.ds(p0, _PW)],
            idx_v.at[b // 2, pl.ds((b % 2) * _PW, _PW)],
            sem_in))
    copies.append(pltpu.async_copy(pos_hbm.at[pl.ds(p0, _PW), :], pos_v,
                                   sem_in))
    for c in copies:
        c.wait()

    gathers = []
    for g in range(_NG):
        gathers.append(pltpu.async_copy(
            table_hbm.at[idx_v.at[g]],
            rows_v.at[pl.ds(g * _GC, _GC)],
            sem_g))
    for c in gathers:
        c.wait()

    def row_body(l, carry):
        pj = [pos_v[l, pl.ds(16 * j, 16)] for j in range(4)]
        for b in range(_B):
            r = b * _PW + l
            for j in range(4):
                rows_v[r, pl.ds(16 * j, 16)] = (
                    rows_v[r, pl.ds(16 * j, 16)] * _SCALE + pj[j])
        return carry
    lax.fori_loop(0, _PW, row_body, 0)

    outs = []
    for b in range(_B):
        outs.append(pltpu.async_copy(
            rows_v.at[pl.ds(b * _PW, _PW)],
            out_hbm.at[b, pl.ds(p0, _PW), :],
            sem_out))
    for c in outs:
        c.wait()


@jax.jit
def kernel(x, table):
    x32 = x.astype(jnp.int32)
    pos = jnp.asarray(_POS)
    mesh = plsc.VectorSubcoreMesh(core_axis_name="c", subcore_axis_name="s")
    out = pl.kernel(
        _sc_body,
        out_type=jax.ShapeDtypeStruct((_B, _L, _D), jnp.float32),
        mesh=mesh,
        compiler_params=pltpu.CompilerParams(use_tc_tiling_on_sc=False),
        scratch_types=[
            pltpu.VMEM((_NG, _GC), jnp.int32),
            pltpu.VMEM((_RW, _D), jnp.float32),
            pltpu.VMEM((_PW, _D), jnp.float32),
            pltpu.SemaphoreType.DMA,
            pltpu.SemaphoreType.DMA,
            pltpu.SemaphoreType.DMA,
        ],
    )(x32, table, pos)
    return out


# speedup vs baseline: 1.5307x; 1.5307x over previous
import numpy as np
import jax
import jax.numpy as jnp
from jax import lax
from jax.experimental import pallas as pl
from jax.experimental.pallas import tpu as pltpu
from jax.experimental.pallas import tpu_sc as plsc

_V = 1000000
_D = 64
_B = 16
_L = 2048
_NC = 2
_NW = 32
_PW = _L // _NW
_RW = _B * _PW
_C = 32
_NCH = _RW // _C
_SCALE = 8.0


def _pos_encoding_np():
    pos = np.arange(_L)[:, None]
    i = np.arange(_D)[None, :]
    angle_rates = 1.0 / np.power(10000, 2 * (i // 2) / np.float32(_D))
    angle_rads = pos * angle_rates
    angle_rads[:, 0::2] = np.sin(angle_rads[:, 0::2])
    angle_rads[:, 1::2] = np.cos(angle_rads[:, 1::2])
    return angle_rads.astype(np.float32)


_POS = _pos_encoding_np()


def _sc_body(xt_hbm, table_hbm, pos_hbm, out_hbm,
             idx_v, rows_v, out_v, pos_v, sem_in, sem_g, sem_out):
    wid = lax.axis_index("s") * _NC + lax.axis_index("c")
    p0 = pl.multiple_of(wid * _PW, _PW)

    c_idx = pltpu.async_copy(xt_hbm.at[wid], idx_v, sem_in)
    c_pos = pltpu.async_copy(pos_hbm.at[wid], pos_v, sem_in)
    c_idx.wait()
    c_pos.wait()

    def chunk_body(c, carry):
        b = lax.div(c, 2)
        l0 = lax.rem(c, 2) * _C
        q = lax.rem(c, 4) * _C
        row = lax.div(c, 4)
        copies = []
        for h in range(2):
            iv = idx_v[row, pl.ds(q + 16 * h, 16)]
            for k in range(16):
                t = iv[k]
                copies.append(pltpu.async_copy(
                    table_hbm.at[t], rows_v.at[16 * h + k], sem_g))
        for cp in copies:
            cp.wait()
        for r in range(_C):
            for j in range(4):
                out_v[r, pl.ds(16 * j, 16)] = (
                    rows_v[r, pl.ds(16 * j, 16)] * _SCALE
                    + pos_v[l0 + r, pl.ds(16 * j, 16)])
        pltpu.async_copy(
            out_v, out_hbm.at[b, pl.ds(p0 + l0, _C), :], sem_out).wait()
        return carry
    lax.fori_loop(0, _NCH, chunk_body, 0)


@jax.jit
def kernel(x, table):
    x32 = x.astype(jnp.int32)
    xt = (x32.reshape(_B, _NW, _PW).transpose(1, 0, 2)
          .reshape(_NW, 8, 128))
    posw = jnp.asarray(_POS.reshape(_NW, _PW, _D))
    mesh = plsc.VectorSubcoreMesh(core_axis_name="c", subcore_axis_name="s")
    out = pl.kernel(
        _sc_body,
        out_type=jax.ShapeDtypeStruct((_B, _L, _D), jnp.float32),
        mesh=mesh,
        compiler_params=pltpu.CompilerParams(
            use_tc_tiling_on_sc=True, needs_layout_passes=False),
        scratch_types=[
            pltpu.VMEM((8, 128), jnp.int32),
            pltpu.VMEM((_C, _D), jnp.float32),
            pltpu.VMEM((_C, _D), jnp.float32),
            pltpu.VMEM((_PW, _D), jnp.float32),
            pltpu.SemaphoreType.DMA,
            pltpu.SemaphoreType.DMA,
            pltpu.SemaphoreType.DMA,
        ],
    )(xt, table, posw)
    return out


# speedup vs baseline: 1.5312x; 1.0003x over previous
import numpy as np
import jax
import jax.numpy as jnp
from jax import lax
from jax.experimental import pallas as pl
from jax.experimental.pallas import tpu as pltpu
from jax.experimental.pallas import tpu_sc as plsc

_V = 1000000
_D = 64
_B = 16
_L = 2048
_NC = 2
_NW = 32
_PW = _L // _NW
_RW = _B * _PW
_C = 32
_NCH = _RW // _C
_SCALE = 8.0


def _pos_encoding_np():
    pos = np.arange(_L)[:, None]
    i = np.arange(_D)[None, :]
    angle_rates = 1.0 / np.power(10000, 2 * (i // 2) / np.float32(_D))
    angle_rads = pos * angle_rates
    angle_rads[:, 0::2] = np.sin(angle_rads[:, 0::2])
    angle_rads[:, 1::2] = np.cos(angle_rads[:, 1::2])
    return angle_rads.astype(np.float32)


_POS = _pos_encoding_np()


def _sc_body(xt_hbm, table_hbm, pos_hbm, out_hbm,
             idx_v, rows_v, out_v, pos_v, sem_in, sem_g, sem_out):
    wid = lax.axis_index("s") * _NC + lax.axis_index("c")
    p0 = pl.multiple_of(wid * _PW, _PW)

    c_idx = pltpu.async_copy(xt_hbm.at[wid], idx_v, sem_in)
    c_pos = pltpu.async_copy(pos_hbm.at[wid], pos_v, sem_in)
    c_idx.wait()
    c_pos.wait()

    def chunk_body(c, carry):
        b = lax.div(c, 2)
        l0 = lax.rem(c, 2) * _C
        q = lax.rem(c, 4) * _C
        row = lax.div(c, 4)
        copies = []
        for h in range(2):
            iv = idx_v[row, pl.ds(q + 16 * h, 16)]
            for k in range(16):
                t = iv[k]
                copies.append(pltpu.async_copy(
                    table_hbm.at[t], rows_v.at[16 * h + k], sem_g))
        for cp in copies:
            cp.wait()
        for r in range(_C):
            for j in range(4):
                out_v[r, pl.ds(16 * j, 16)] = (
                    rows_v[r, pl.ds(16 * j, 16)] * _SCALE
                    + pos_v[l0 + r, pl.ds(16 * j, 16)])
        pltpu.async_copy(
            out_v, out_hbm.at[b, pl.ds(p0 + l0, _C), :], sem_out).wait()
        return carry
    lax.fori_loop(0, _NCH, chunk_body, 0)


@jax.jit
def kernel(x, table):
    x32 = x.astype(jnp.int32)
    xt = (x32.reshape(_B, _NW, _PW).transpose(1, 0, 2)
          .reshape(_NW, 8, 128))
    posw = jnp.asarray(_POS.reshape(_NW, _PW, _D))
    mesh = plsc.VectorSubcoreMesh(core_axis_name="c", subcore_axis_name="s")
    out = pl.kernel(
        _sc_body,
        out_type=jax.ShapeDtypeStruct((_B, _L, _D), jnp.float32),
        mesh=mesh,
        compiler_params=pltpu.CompilerParams(use_tc_tiling_on_sc=True),
        scratch_types=[
            pltpu.VMEM((8, 128), jnp.int32),
            pltpu.VMEM((_C, _D), jnp.float32),
            pltpu.VMEM((_C, _D), jnp.float32),
            pltpu.VMEM((_PW, _D), jnp.float32),
            pltpu.SemaphoreType.DMA,
            pltpu.SemaphoreType.DMA,
            pltpu.SemaphoreType.DMA,
        ],
    )(xt, table, posw)
    return out
